# Initial kernel scaffold; baseline (speedup 1.0000x reference)
import jax
import jax.numpy as jnp
from jax.experimental import pallas as pl


def kernel(x, edge_index, t1_W, t1_b, att_l0, att_r0, att_l1, att_r1, t2_W, t2_b):
    raise NotImplementedError("write your pallas kernel here")


# speedup vs baseline: 19.0878x; 19.0878x over previous
import functools

import jax
import jax.numpy as jnp
from jax import lax
from jax.experimental import pallas as pl
from jax.experimental.pallas import tpu as pltpu
from jax.experimental.pallas import tpu_sc as plsc

N = 10000
E = 320000
D = 128
NCLS = 40
EPSV = 0.3

NC = 2
NS = 16
CHUNK = 80
EPT = E // (NC * NS)
STEPS = EPT // CHUNK
RPT = 624
TAIL = N - RPT * NS
ZROWS = 104


def _zero_vec(ref, n):
    z = jnp.zeros((16,), ref.dtype)

    def body(i, _):
        ref[pl.ds(i * 16, 16)] = z
        return 0

    lax.fori_loop(0, n // 16, body, 0)


def _zero_rows(ref, rows):
    z = jnp.zeros((16,), jnp.float32)

    def body(i, _):
        for f in range(D // 16):
            ref[i, pl.ds(f * 16, 16)] = z
        return 0

    lax.fori_loop(0, rows, body, 0)


def _deg_body(dst_hbm, out0_hbm, out1_hbm, idx_v, ones_v, zbuf_v, acc_sp):
    c = lax.axis_index("c")
    s = lax.axis_index("s")

    for j in range(CHUNK // 16):
        ones_v[pl.ds(j * 16, 16)] = jnp.ones((16,), jnp.float32)
    _zero_vec(zbuf_v, 640)

    pltpu.sync_copy(zbuf_v.at[pl.ds(0, 624)], acc_sp.at[pl.ds(s * 624, 624)])

    @pl.when(s == NS - 1)
    def _():
        pltpu.sync_copy(zbuf_v.at[pl.ds(0, 16)], acc_sp.at[pl.ds(9984, 16)])

    plsc.subcore_barrier()

    base = c * (E // NC) + s * EPT

    def step(i, _):
        pltpu.sync_copy(dst_hbm.at[pl.ds(base + i * CHUNK, CHUNK)], idx_v)
        pltpu.sync_copy(ones_v, acc_sp.at[idx_v], add=True)
        return 0

    lax.fori_loop(0, STEPS, step, 0)
    plsc.subcore_barrier()

    @pl.when((s == 0) & (c == 0))
    def _():
        pltpu.sync_copy(acc_sp, out0_hbm)

    @pl.when((s == 0) & (c == 1))
    def _():
        pltpu.sync_copy(acc_sp, out1_hbm)


def _deg_call(dst):
    f = pl.kernel(
        _deg_body,
        out_type=[jax.ShapeDtypeStruct((N,), jnp.float32),
                  jax.ShapeDtypeStruct((N,), jnp.float32)],
        mesh=plsc.VectorSubcoreMesh(core_axis_name="c", subcore_axis_name="s"),
        compiler_params=pltpu.CompilerParams(needs_layout_passes=False),
        scratch_types=[
            pltpu.VMEM((CHUNK,), jnp.int32),
            pltpu.VMEM((CHUNK,), jnp.float32),
            pltpu.VMEM((640,), jnp.float32),
            pltpu.VMEM_SHARED((N,), jnp.float32),
        ],
    )
    return f(dst)


def _layer_body(g_hbm, al_hbm, ar_hbm, src_hbm, dst_hbm, out0_hbm, out1_hbm,
                al_v, ar_v, src_v, dst_v, w_v, rows_v, zrow_v, acc_sp, sem):
    c = lax.axis_index("c")
    s = lax.axis_index("s")

    _zero_rows(zrow_v, ZROWS)
    pltpu.sync_copy(al_hbm, al_v)
    pltpu.sync_copy(ar_hbm, ar_v)
    for b in range(RPT // ZROWS):
        pltpu.sync_copy(zrow_v, acc_sp.at[pl.ds(s * RPT + b * ZROWS, ZROWS)])

    @pl.when(s == NS - 1)
    def _():
        pltpu.sync_copy(zrow_v.at[pl.ds(0, TAIL)],
                        acc_sp.at[pl.ds(N - TAIL, TAIL)])

    plsc.subcore_barrier()

    base = c * (E // NC) + s * EPT

    def step(i, _):
        off = base + i * CHUNK
        pltpu.sync_copy(src_hbm.at[pl.ds(off, CHUNK)], src_v)
        pltpu.sync_copy(dst_hbm.at[pl.ds(off, CHUNK)], dst_v)
        cp = pltpu.async_copy(g_hbm.at[src_v], rows_v, sem)
        for j in range(CHUNK // 16):
            si = src_v[pl.ds(j * 16, 16)]
            di = dst_v[pl.ds(j * 16, 16)]
            a = plsc.load_gather(al_v, [si])
            b = plsc.load_gather(ar_v, [di])
            z = jnp.clip(a + b, -15.0, 15.0)
            t = 1.0 - 2.0 / (jnp.exp(2.0 * z) + 1.0)
            w_v[pl.ds(j * 16, 16)] = t
        cp.wait()

        def scale(r, _):
            wb = plsc.load_gather(w_v, [jnp.broadcast_to(r, (16,))])
            for f in range(D // 16):
                rows_v[r, pl.ds(f * 16, 16)] = (
                    rows_v[r, pl.ds(f * 16, 16)] * wb)
            return 0

        lax.fori_loop(0, CHUNK, scale, 0, unroll=4)
        pltpu.sync_copy(rows_v, acc_sp.at[dst_v], add=True)
        return 0

    lax.fori_loop(0, STEPS, step, 0)
    plsc.subcore_barrier()

    @pl.when(c == 0)
    def _():
        pltpu.sync_copy(acc_sp.at[pl.ds(s * RPT, RPT)],
                        out0_hbm.at[pl.ds(s * RPT, RPT)])

    @pl.when(c == 1)
    def _():
        pltpu.sync_copy(acc_sp.at[pl.ds(s * RPT, RPT)],
                        out1_hbm.at[pl.ds(s * RPT, RPT)])

    @pl.when((s == NS - 1) & (c == 0))
    def _():
        pltpu.sync_copy(acc_sp.at[pl.ds(N - TAIL, TAIL)],
                        out0_hbm.at[pl.ds(N - TAIL, TAIL)])

    @pl.when((s == NS - 1) & (c == 1))
    def _():
        pltpu.sync_copy(acc_sp.at[pl.ds(N - TAIL, TAIL)],
                        out1_hbm.at[pl.ds(N - TAIL, TAIL)])


def _layer_call(g, al, ar, src, dst):
    f = pl.kernel(
        _layer_body,
        out_type=[jax.ShapeDtypeStruct((N, D), jnp.float32),
                  jax.ShapeDtypeStruct((N, D), jnp.float32)],
        mesh=plsc.VectorSubcoreMesh(core_axis_name="c", subcore_axis_name="s"),
        compiler_params=pltpu.CompilerParams(needs_layout_passes=False),
        scratch_types=[
            pltpu.VMEM((N,), jnp.float32),
            pltpu.VMEM((N,), jnp.float32),
            pltpu.VMEM((CHUNK,), jnp.int32),
            pltpu.VMEM((CHUNK,), jnp.int32),
            pltpu.VMEM((CHUNK,), jnp.float32),
            pltpu.VMEM((CHUNK, D), jnp.float32),
            pltpu.VMEM((ZROWS, D), jnp.float32),
            pltpu.VMEM_SHARED((N, D), jnp.float32),
            pltpu.SemaphoreType.DMA,
        ],
    )
    return f(g, al, ar, src, dst)


def _tc_a_body(x_ref, w1_ref, b1_ref, attl_ref, attr_ref, dp0_ref, dp1_ref,
               h_ref, g_ref, dinv_ref, al_ref, ar_ref):
    cdims = (((1,), (1,)), ((), ()))
    xw = lax.dot_general(x_ref[...], w1_ref[...], cdims,
                         preferred_element_type=jnp.float32)
    h = jnp.maximum(xw + b1_ref[...], 0.0)
    dinv = lax.rsqrt(dp0_ref[...] + dp1_ref[...] + 1.0)
    vdims = (((1,), (0,)), ((), ()))
    h_ref[...] = h
    g_ref[...] = h * dinv
    dinv_ref[...] = dinv
    al_ref[...] = lax.dot_general(h, attl_ref[...], vdims,
                                  preferred_element_type=jnp.float32)
    ar_ref[...] = lax.dot_general(h, attr_ref[...], vdims,
                                  preferred_element_type=jnp.float32)


def _tc_b_body(a0_ref, a1_ref, h0_ref, dinv_ref, al_ref, ar_ref,
               attl_ref, attr_ref, h1_ref, g1_ref, al1_ref, ar1_ref):
    dinv = dinv_ref[...]
    t = jnp.tanh(al_ref[...] + ar_ref[...])
    h1 = (dinv * (a0_ref[...] + a1_ref[...])
          + (t * dinv * dinv + EPSV) * h0_ref[...])
    vdims = (((1,), (0,)), ((), ()))
    h1_ref[...] = h1
    g1_ref[...] = h1 * dinv
    al1_ref[...] = lax.dot_general(h1, attl_ref[...], vdims,
                                   preferred_element_type=jnp.float32)
    ar1_ref[...] = lax.dot_general(h1, attr_ref[...], vdims,
                                   preferred_element_type=jnp.float32)


def _tc_c_body(a0_ref, a1_ref, h1_ref, h0_ref, dinv_ref, al_ref, ar_ref,
               w2_ref, b2_ref, out_ref):
    dinv = dinv_ref[...]
    t = jnp.tanh(al_ref[...] + ar_ref[...])
    h2 = (dinv * (a0_ref[...] + a1_ref[...])
          + t * dinv * dinv * h1_ref[...] + EPSV * h0_ref[...])
    cdims = (((1,), (1,)), ((), ()))
    logits = lax.dot_general(h2, w2_ref[...], cdims,
                             preferred_element_type=jnp.float32) + b2_ref[...]
    m = jnp.max(logits, axis=1, keepdims=True)
    ex = jnp.exp(logits - m)
    lse = jnp.log(jnp.sum(ex, axis=1, keepdims=True)) + m
    out_ref[...] = logits - lse


def _sds(shape):
    return jax.ShapeDtypeStruct(shape, jnp.float32)


def kernel(x, edge_index, t1_W, t1_b, att_l0, att_r0, att_l1, att_r1,
           t2_W, t2_b):
    src = edge_index[0]
    dst = edge_index[1]
    b1 = t1_b.reshape(1, D)
    b2 = t2_b.reshape(1, NCLS)

    d0, d1 = _deg_call(dst)
    dp0 = d0.reshape(N, 1)
    dp1 = d1.reshape(N, 1)

    h0, g0, dinv, al0, ar0 = pl.pallas_call(
        _tc_a_body,
        out_shape=[_sds((N, D)), _sds((N, D)), _sds((N, 1)),
                   _sds((N, 1)), _sds((N, 1))],
    )(x, t1_W, b1, att_l0, att_r0, dp0, dp1)

    a00, a01 = _layer_call(g0, al0.reshape(N), ar0.reshape(N), src, dst)

    h1, g1, al1, ar1 = pl.pallas_call(
        _tc_b_body,
        out_shape=[_sds((N, D)), _sds((N, D)), _sds((N, 1)), _sds((N, 1))],
    )(a00, a01, h0, dinv, al0, ar0, att_l1, att_r1)

    a10, a11 = _layer_call(g1, al1.reshape(N), ar1.reshape(N), src, dst)

    out = pl.pallas_call(
        _tc_c_body,
        out_shape=_sds((N, NCLS)),
    )(a10, a11, h1, h0, dinv, al1, ar1, t2_W, b2)
    return out
